# baseline (speedup 1.0000x reference)
import jax
import jax.numpy as jnp
from jax import lax
from jax.experimental import pallas as pl
from jax.experimental.pallas import tpu as pltpu
from jax.experimental.pallas import tpu_sc as plsc

B, N, M = 16, 2048, 2048
NP1, MP1 = N + 1, M + 1
LANES = 16
HALF = N // 2
NGRP = HALF // LANES
CH = 16
NCH = HALF // CH


def _body(scores3d, gt0, gt1, out,
          g0_v, g1_v, dust_v, rows_v, part_v, stage_v, sh_part, acc_v,
          sem0, sem1):
    c = lax.axis_index("c")
    s = lax.axis_index("s")
    b = c * 8 + s // 2
    h = s % 2
    base_i = h * HALF

    pltpu.sync_copy(gt0.at[b, pl.ds(base_i, HALF)], g0_v)
    pltpu.sync_copy(gt1.at[b, pl.ds(base_i, HALF)], g1_v)
    pltpu.sync_copy(scores3d.at[b, pl.ds(N, 1), pl.ds(base_i, HALF)], dust_v)

    lane_iota = lax.iota(jnp.int32, LANES)
    sems = (sem0, sem1)

    def start(k, slot):
        pltpu.async_copy(
            scores3d.at[b, pl.ds(base_i + k * CH, CH), pl.ds(0, MP1)],
            rows_v.at[slot], sems[slot])

    def wait(slot):
        pltpu.make_async_copy(
            scores3d.at[b, pl.ds(base_i, CH), pl.ds(0, MP1)],
            rows_v.at[slot], sems[slot]).wait()

    def pick(k, slot):
        g0 = g0_v[pl.ds(k * LANES, LANES)]
        ind0 = jnp.where(g0 < 0, g0 + MP1, g0)
        return plsc.load_gather(rows_v.at[slot], [lane_iota, ind0])

    start(0, 0)
    start(1, 1)

    def stream_body(t, tp):
        k = t * 2
        wait(0)
        tp = tp + pick(k, 0)

        @pl.when(k + 2 < NCH)
        def _():
            start(k + 2, 0)
        wait(1)
        tp = tp + pick(k + 1, 1)

        @pl.when(k + 3 < NCH)
        def _():
            start(k + 3, 1)
        return tp

    tp = lax.fori_loop(0, NCH // 2, stream_body,
                       jnp.zeros((LANES,), jnp.float32))

    def tn_body(g, carry):
        tn, cnt = carry
        mask = g1_v[pl.ds(g * LANES, LANES)] == -1
        tn = tn + jnp.where(mask, dust_v[0, pl.ds(g * LANES, LANES)], 0.0)
        cnt = cnt + jnp.where(mask, 1, 0)
        return tn, cnt

    zf = jnp.zeros((LANES,), jnp.float32)
    tn, cnt = lax.fori_loop(0, NGRP, tn_body,
                            (zf, jnp.zeros((LANES,), jnp.int32)))

    tp_s = jnp.sum(tp)
    tn_s = jnp.sum(tn)
    cnt_s = jnp.sum(cnt).astype(jnp.float32)

    sel = (lane_iota == b).astype(jnp.float32)
    part_v[0, :] = sel * tp_s
    part_v[1, :] = sel * tn_s
    part_v[2, :] = sel * cnt_s
    pltpu.sync_copy(part_v, sh_part.at[s])
    plsc.subcore_barrier()

    @pl.when(s == 0)
    def _():
        pltpu.sync_copy(sh_part, acc_v)

        atp, atn, acnt = zf, zf, zf
        for r in range(16):
            atp = atp + acc_v[r, 0, :]
            atn = atn + acc_v[r, 1, :]
            acnt = acnt + acc_v[r, 2, :]
        term = (-atp - atn) / (acnt + jnp.float32(M)) * jnp.float32(1.0 / B)
        stage_v[...] = term
        pltpu.sync_copy(stage_v, out.at[c])


@jax.jit
def _run(scores3d, gt0, gt1):
    kern = pl.kernel(
        _body,
        out_type=jax.ShapeDtypeStruct((2, LANES), jnp.float32),
        mesh=plsc.VectorSubcoreMesh(core_axis_name="c", subcore_axis_name="s"),
        compiler_params=pltpu.CompilerParams(
            use_tc_tiling_on_sc=False, needs_layout_passes=False),
        scratch_types=[
            pltpu.VMEM((HALF,), jnp.int32),
            pltpu.VMEM((HALF,), jnp.int32),
            pltpu.VMEM((1, HALF), jnp.float32),
            pltpu.VMEM((2, CH, MP1), jnp.float32),
            pltpu.VMEM((3, LANES), jnp.float32),
            pltpu.VMEM((LANES,), jnp.float32),
            pltpu.VMEM_SHARED((16, 3, LANES), jnp.float32),
            pltpu.VMEM((16, 3, LANES), jnp.float32),
            pltpu.SemaphoreType.DMA,
            pltpu.SemaphoreType.DMA,
        ],
    )
    return kern(scores3d, gt0, gt1)


def kernel(gt_matches0, gt_matches1, scores):
    g0 = gt_matches0.astype(jnp.int32)
    g1 = gt_matches1.astype(jnp.int32)
    out = _run(scores, g0, g1)
    return jnp.sum(out)


# speedup vs baseline: 13.7068x; 13.7068x over previous
import jax
import jax.numpy as jnp
from jax import lax
from jax.experimental import pallas as pl
from jax.experimental.pallas import tpu as pltpu
from jax.experimental.pallas import tpu_sc as plsc

B, N, M = 16, 2048, 2048
NP1, MP1 = N + 1, M + 1
LANES = 16
HALF = N // 2
NGRP = HALF // LANES


def _body(scores3d, gt0, gt1, out,
          g0_v, g1_v, dust_v, slots_v, part_v, sh_part, acc_v, sem):
    c = lax.axis_index("c")
    s = lax.axis_index("s")
    b = c * 8 + s // 2
    h = s % 2
    base_i = h * HALF

    pltpu.sync_copy(gt0.at[pl.ds(b, 1), pl.ds(base_i, HALF)], g0_v)
    pltpu.sync_copy(gt1.at[pl.ds(b, 1), pl.ds(base_i, HALF)], g1_v)
    pltpu.sync_copy(scores3d.at[pl.ds(b, 1), pl.ds(N, 1), pl.ds(base_i, HALF)],
                    dust_v)

    lane_iota = lax.iota(jnp.int32, LANES)

    DEPTH = 0
    RING = 8

    def fire(g):
        jv = g0_v[0, pl.ds(g * LANES, LANES)]
        iv = jnp.where(jv < 0, jv + MP1, jv)
        segv = iv - (iv % 128)
        i0 = base_i + g * LANES
        r = g % RING
        for l in range(LANES):
            pltpu.async_copy(
                scores3d.at[pl.ds(b, 1), pl.ds(i0 + l, 1),
                            pl.ds(pl.multiple_of(segv[l], 128), 128)],
                slots_v.at[pl.ds(r, 1), pl.ds(l // 8, 1),
                           pl.ds((l % 8) * 128, 128)],
                sem)

    def drain(g):
        r = g % RING
        for l in range(LANES):
            pltpu.make_async_copy(
                scores3d.at[pl.ds(0, 1), pl.ds(0, 1), pl.ds(0, 128)],
                slots_v.at[pl.ds(r, 1), pl.ds(l // 8, 1),
                           pl.ds((l % 8) * 128, 128)],
                sem).wait()

    def process(g, tp):
        jv = g0_v[0, pl.ds(g * LANES, LANES)]
        iv = jnp.where(jv < 0, jv + MP1, jv)
        jt = (iv // LANES) % 8
        jl = iv % LANES
        r = g % RING
        for l in range(LANES):
            seg = slots_v[r, l // 8, pl.ds((l % 8) * 128 + jt[l] * LANES,
                                           LANES)]
            tp = tp + jnp.where(lane_iota == jl[l], seg, 0.0)
        return tp

    zf = jnp.zeros((LANES,), jnp.float32)

    def pipe_body(t, tp):
        @pl.when(t < NGRP)
        def _():
            fire(t)

        @pl.when(t >= DEPTH)
        def _():
            drain(t - DEPTH)
        pg = jnp.maximum(t - DEPTH, 0)
        contrib = process(pg, zf)
        return tp + jnp.where(t >= DEPTH, contrib, 0.0)

    tp = lax.fori_loop(0, NGRP + DEPTH, pipe_body, zf)

    def tn_body(g, carry):
        tn, cnt = carry
        mask = g1_v[0, pl.ds(g * LANES, LANES)] == -1
        tn = tn + jnp.where(mask, dust_v[0, 0, pl.ds(g * LANES, LANES)], 0.0)
        cnt = cnt + jnp.where(mask, 1, 0)
        return tn, cnt

    tn, cnt = lax.fori_loop(0, NGRP, tn_body,
                            (zf, jnp.zeros((LANES,), jnp.int32)))

    tp_s = jnp.sum(tp)
    tn_s = jnp.sum(tn)
    cnt_s = jnp.sum(cnt).astype(jnp.float32)

    sel = (lane_iota == b).astype(jnp.float32)
    part_v[0, 0, pl.ds(0, LANES)] = sel * tp_s
    part_v[0, 1, pl.ds(0, LANES)] = sel * tn_s
    part_v[0, 2, pl.ds(0, LANES)] = sel * cnt_s
    pltpu.sync_copy(part_v, sh_part.at[pl.ds(s, 1)])
    plsc.subcore_barrier()

    @pl.when(s == 0)
    def _():
        pltpu.sync_copy(sh_part, acc_v)
        atp, atn, acnt = zf, zf, zf
        for r in range(16):
            atp = atp + acc_v[r, 0, pl.ds(0, LANES)]
            atn = atn + acc_v[r, 1, pl.ds(0, LANES)]
            acnt = acnt + acc_v[r, 2, pl.ds(0, LANES)]
        term = (-atp - atn) / (acnt + jnp.float32(M)) * jnp.float32(1.0 / B)
        part_v[0, 0, pl.ds(0, LANES)] = term
        pltpu.sync_copy(part_v.at[pl.ds(0, 1), pl.ds(0, 1), :],
                        out.at[pl.ds(c, 1)])


@jax.jit
def _run(scores3d, gt0, gt1):
    kern = pl.kernel(
        _body,
        out_type=jax.ShapeDtypeStruct((2, 1, 1024), jnp.float32),
        mesh=plsc.VectorSubcoreMesh(core_axis_name="c", subcore_axis_name="s"),
        compiler_params=pltpu.CompilerParams(needs_layout_passes=False),
        scratch_types=[
            pltpu.VMEM((1, HALF), jnp.int32),
            pltpu.VMEM((1, HALF), jnp.int32),
            pltpu.VMEM((1, 1, HALF), jnp.float32),
            pltpu.VMEM((8, 2, 1024), jnp.float32),
            pltpu.VMEM((1, 3, 1024), jnp.float32),
            pltpu.VMEM_SHARED((16, 3, 1024), jnp.float32),
            pltpu.VMEM((16, 3, 1024), jnp.float32),
            pltpu.SemaphoreType.DMA,
        ],
    )
    return kern(scores3d, gt0, gt1)


def kernel(gt_matches0, gt_matches1, scores):
    g0 = gt_matches0.astype(jnp.int32)
    g1 = gt_matches1.astype(jnp.int32)
    out = _run(scores, g0, g1)
    return jnp.sum(out[:, :, :LANES])
